# baseline (speedup 1.0000x reference)
import functools
import math

import jax
import jax.numpy as jnp
from jax.experimental import pallas as pl
from jax.experimental.pallas import tpu as pltpu

_PAD = 128


def _mm_body(act, x_ref, w_ref, b_ref, o_ref):
    y = jnp.dot(x_ref[...], w_ref[...], preferred_element_type=jnp.float32) + b_ref[...]
    if act == 'silu':
        y = y * jax.nn.sigmoid(y)
    o_ref[...] = y


def _mm(x, w, b=None, act=None, bm=512):
    M, K = x.shape
    N = w.shape[1]
    b2 = jnp.zeros((1, N), jnp.float32) if b is None else b.reshape(1, N)
    return pl.pallas_call(
        functools.partial(_mm_body, act),
        out_shape=jax.ShapeDtypeStruct((M, N), jnp.float32),
        grid=(M // bm,),
        in_specs=[pl.BlockSpec((bm, K), lambda i: (i, 0)),
                  pl.BlockSpec((K, N), lambda i: (0, 0)),
                  pl.BlockSpec((1, N), lambda i: (0, 0))],
        out_specs=pl.BlockSpec((bm, N), lambda i: (i, 0)),
        compiler_params=pltpu.CompilerParams(
            dimension_semantics=("arbitrary",),
            vmem_limit_bytes=100 * 1024 * 1024,
        ),
        name=f"mm_{N}",
    )(x, w, b2)


def _attn_chain_body(Wd, nlayers, x_ref, wqkv_ref, bqkv_ref, wdw_ref, bdw_ref,
                     temp_ref, wproj_ref, bproj_ref, o_ref, qpad_ref):
    n, C = x_ref.shape[1], x_ref.shape[2]
    C3 = 3 * C
    hd = C // 4

    qpad_ref[:_PAD] = jnp.zeros((_PAD, C3), jnp.float32)
    qpad_ref[_PAD + n:] = jnp.zeros((_PAD, C3), jnp.float32)

    rx = jax.lax.broadcasted_iota(jnp.int32, (n // Wd, Wd, C3), 1).reshape(n, C3)
    mp = jnp.where(rx < Wd - 2, 1.0, 0.0)
    mm_ = jnp.where(rx >= 2, 1.0, 0.0)

    y = x_ref[0]
    for l in range(nlayers):
        qkv = jax.lax.dot_general(y, wqkv_ref[l], (((1,), (1,)), ((), ())),
                                  preferred_element_type=jnp.float32) + bqkv_ref[l]
        qpad_ref[_PAD:_PAD + n] = qkv
        wd = wdw_ref[l]
        acc0 = jnp.zeros((n, C3), jnp.float32)
        accm = jnp.zeros((n, C3), jnp.float32)
        accp = jnp.zeros((n, C3), jnp.float32)
        for ky in range(3):
            off = (ky - 1) * 2 * Wd
            accm = accm + wd[3 * ky:3 * ky + 1] * qpad_ref[_PAD + off - 2:_PAD + off - 2 + n]
            acc0 = acc0 + wd[3 * ky + 1:3 * ky + 2] * qpad_ref[_PAD + off:_PAD + off + n]
            accp = accp + wd[3 * ky + 2:3 * ky + 3] * qpad_ref[_PAD + off + 2:_PAD + off + 2 + n]
        conv = acc0 + mm_ * accm + mp * accp + bdw_ref[l]

        outs = []
        for h in range(4):
            qh = conv[:, h * hd:(h + 1) * hd]
            kh = conv[:, C + h * hd:C + (h + 1) * hd]
            vh = conv[:, 2 * C + h * hd:2 * C + (h + 1) * hd]
            qn = qh * jax.lax.rsqrt(jnp.maximum(jnp.sum(qh * qh, axis=0, keepdims=True), 1e-24))
            kn = kh * jax.lax.rsqrt(jnp.maximum(jnp.sum(kh * kh, axis=0, keepdims=True), 1e-24))
            att = jax.lax.dot_general(qn, kn, (((0,), (0,)), ((), ())),
                                      preferred_element_type=jnp.float32)
            att = att * temp_ref[l, h]
            att = att - jnp.max(att, axis=-1, keepdims=True)
            att = jnp.exp(att)
            att = att / jnp.sum(att, axis=-1, keepdims=True)
            outs.append(jax.lax.dot_general(vh, att, (((1,), (1,)), ((), ())),
                                            preferred_element_type=jnp.float32))
        hall = jnp.concatenate(outs, axis=-1)
        y = jax.lax.dot_general(hall, wproj_ref[l], (((1,), (1,)), ((), ())),
                                preferred_element_type=jnp.float32) + bproj_ref[l]
    o_ref[0] = y


def _attn_chain(x, aps, Wd):
    B, n, C = x.shape
    C3 = 3 * C
    L = len(aps)
    wqkv = jnp.stack([ap['qkv_w'][:, :, 0, 0] for ap in aps])
    bqkv = jnp.stack([ap['qkv_b'][None, :] for ap in aps])
    wdw = jnp.stack([ap['dw_w'][:, 0].reshape(C3, 9).T for ap in aps])
    bdw = jnp.stack([ap['dw_b'][None, :] for ap in aps])
    temp = jnp.stack([ap['temp'][:, 0, 0] for ap in aps])
    wproj = jnp.stack([ap['proj_w'][:, :, 0, 0] for ap in aps])
    bproj = jnp.stack([ap['proj_b'][None, :] for ap in aps])

    body = functools.partial(_attn_chain_body, Wd, L)
    return pl.pallas_call(
        body,
        out_shape=jax.ShapeDtypeStruct((B, n, C), jnp.float32),
        grid=(B,),
        in_specs=[
            pl.BlockSpec((1, n, C), lambda b: (b, 0, 0)),
            pl.BlockSpec((L, C3, C), lambda b: (0, 0, 0)),
            pl.BlockSpec((L, 1, C3), lambda b: (0, 0, 0)),
            pl.BlockSpec((L, 9, C3), lambda b: (0, 0, 0)),
            pl.BlockSpec((L, 1, C3), lambda b: (0, 0, 0)),
            pl.BlockSpec((L, 4), lambda b: (0, 0), memory_space=pltpu.SMEM),
            pl.BlockSpec((L, C, C), lambda b: (0, 0, 0)),
            pl.BlockSpec((L, 1, C), lambda b: (0, 0, 0)),
        ],
        out_specs=pl.BlockSpec((1, n, C), lambda b: (b, 0, 0)),
        scratch_shapes=[pltpu.VMEM((n + 2 * _PAD, C3), jnp.float32)],
        compiler_params=pltpu.CompilerParams(
            dimension_semantics=("arbitrary",),
            vmem_limit_bytes=110 * 1024 * 1024,
        ),
        name=f"attn_chain_{Wd}",
    )(x, wqkv, bqkv, wdw, bdw, temp, wproj, bproj)


def _gn(x, w, b, groups, eps=1e-5):
    B, H, W, C = x.shape
    xg = x.reshape(B, H * W, groups, C // groups)
    m = xg.mean((1, 3), keepdims=True)
    v = ((xg - m) ** 2).mean((1, 3), keepdims=True)
    xg = (xg - m) * jax.lax.rsqrt(v + eps)
    return xg.reshape(B, H, W, C) * w + b


def _lsp(x, p):
    B, H, W, C = x.shape
    local = jax.lax.conv_general_dilated(
        x, p['dw_w'].transpose(2, 3, 1, 0), (1, 1), [(1, 1), (1, 1)],
        feature_group_count=C, dimension_numbers=('NHWC', 'HWIO', 'NHWC'))
    local = local + p['dw_b']
    x2d = x.reshape(B * H * W, C)
    g1 = _mm(x2d, p['g1_w'][:, :, 0, 0].T, p['g1_b'], act='silu')
    gate = jax.nn.sigmoid(g1 @ p['g2_w'][:, :, 0, 0].T + p['g2_b'])
    lg = local.reshape(B * H * W, C) * gate
    out = _mm(lg, p['pw_w'][:, :, 0, 0].T, p['pw_b']).reshape(B, H, W, C)
    return jax.nn.silu(_gn(out, p['gn_w'], p['gn_b'], 4)) + x


def _avgpool(x, k):
    B, H, W, C = x.shape
    return x.reshape(B, H // k, k, W // k, k, C).mean((2, 4))


def _prca(x, p):
    B, H, W, C = x.shape
    outs = []
    for i in range(len(p['attn'])):
        o = x if i == 0 else _avgpool(x, 2 ** i)
        Wd = W // (2 ** i)
        o3 = _attn_chain(o.reshape(B, Wd * Wd, C), p['attn'][i], Wd)
        o = o3.reshape(B, Wd, Wd, C)
        if i > 0:
            o = jax.image.resize(o, (B, H, W, C), 'bilinear')
        outs.append(o)
    cat = jnp.concatenate([o.reshape(B * H * W, C) for o in outs], axis=-1)
    out = _mm(cat, p['proj_w'][:, :, 0, 0].T, p['proj_b'])
    return out.reshape(B, H, W, C)


_Q = 64
_NST = 16


def _scan_body(dt_ref, dtx_ref, xc_ref, z_ref, bmd_ref, cmd_ref,
               rep_ref, arep_ref, dvec_ref, ow_ref, o_ref, hc_ref, H_ref):
    c = pl.program_id(1)

    @pl.when(c == 0)
    def _():
        hc_ref[...] = jnp.zeros_like(hc_ref)

    dtc = dt_ref[0]
    dt_bc = jnp.dot(rep_ref[...], dtc, preferred_element_type=jnp.float32)
    a = jnp.exp(dt_bc * arep_ref[...])
    bmat = jnp.dot(bmd_ref[0, 0], dtx_ref[0], preferred_element_type=jnp.float32)

    h = hc_ref[...]
    for s in range(_Q):
        h = a[s * _NST:(s + 1) * _NST] * h + bmat[s * _NST:(s + 1) * _NST]
        H_ref[s * _NST:(s + 1) * _NST, :] = h
    hc_ref[...] = h

    y = jnp.dot(cmd_ref[0, 0], H_ref[...], preferred_element_type=jnp.float32)
    y = y + xc_ref[0] * dvec_ref[...]
    z = z_ref[0]
    y = y * (z * jax.nn.sigmoid(z))
    o_ref[0] = jnp.dot(y, ow_ref[...], preferred_element_type=jnp.float32)


def _mamba_scan_pallas(dt, xc, z, Bm, Cm, A, Dvec, out_w):
    B, L, Di = dt.shape
    N = Bm.shape[-1]
    Q = _Q
    NC = L // Q
    QN = Q * N

    eyeQ = jnp.eye(Q, dtype=jnp.float32)
    Bm_r = Bm.reshape(B, NC, Q, N)
    Cm_r = Cm.reshape(B, NC, Q, N)
    bmdiag = (Bm_r[:, :, :, :, None] * eyeQ[None, None, :, None, :]).reshape(B, NC, QN, Q)
    cmdiag = (Cm_r[:, :, :, None, :] * eyeQ[None, None, :, :, None]).reshape(B, NC, Q, QN)
    rep = jnp.repeat(eyeQ, N, axis=0)
    arep = jnp.tile(A.T, (Q, 1))
    dtx = dt * xc
    dvec = Dvec[None, :]
    owT = out_w.T
    D = out_w.shape[0]

    grid = (B, NC)
    seq3 = pl.BlockSpec((1, Q, Di), lambda b, c: (b, c, 0))
    out = pl.pallas_call(
        _scan_body,
        out_shape=jax.ShapeDtypeStruct((B, L, D), jnp.float32),
        grid=grid,
        in_specs=[
            seq3,
            seq3,
            seq3,
            seq3,
            pl.BlockSpec((1, 1, QN, Q), lambda b, c: (b, c, 0, 0)),
            pl.BlockSpec((1, 1, Q, QN), lambda b, c: (b, c, 0, 0)),
            pl.BlockSpec((QN, Q), lambda b, c: (0, 0)),
            pl.BlockSpec((QN, Di), lambda b, c: (0, 0)),
            pl.BlockSpec((1, Di), lambda b, c: (0, 0)),
            pl.BlockSpec((Di, D), lambda b, c: (0, 0)),
        ],
        out_specs=pl.BlockSpec((1, Q, D), lambda b, c: (b, c, 0)),
        scratch_shapes=[
            pltpu.VMEM((N, Di), jnp.float32),
            pltpu.VMEM((QN, Di), jnp.float32),
        ],
        compiler_params=pltpu.CompilerParams(
            dimension_semantics=("arbitrary", "arbitrary"),
            vmem_limit_bytes=100 * 1024 * 1024,
        ),
        name="mamba_scan",
    )(dt, dtx, xc, z, bmdiag, cmdiag, rep, arep, dvec, owT)
    return out


def _mamba_long(x, p, N=16):
    B, L, D = x.shape
    xz = _mm(x.reshape(B * L, D), p['in_proj_w'].T).reshape(B, L, -1)
    Di = xz.shape[-1] // 2
    xin, z = xz[..., :Di], xz[..., Di:]
    kw = p['conv_w'].shape[-1]
    xc = jax.lax.conv_general_dilated(xin, p['conv_w'].transpose(2, 1, 0), (1,), [(kw - 1, 0)],
                                      feature_group_count=Di, dimension_numbers=('NWC', 'WIO', 'NWC'))
    xc = jax.nn.silu(xc + p['conv_b'])
    R = p['dt_proj_w'].shape[1]
    proj = xc @ p['x_proj_w'].T
    dt = jax.nn.softplus(proj[..., :R] @ p['dt_proj_w'].T + p['dt_proj_b'])
    Bm, Cm = proj[..., R:R + N], proj[..., R + N:]
    A = -jnp.exp(p['A_log'])
    return _mamba_scan_pallas(dt, xc, z, Bm, Cm, A, p['D'], p['out_proj_w'])


def _mamba_short(x, p, N=16):
    B, L, D = x.shape
    xz = _mm(x.reshape(B * L, D), p['in_proj_w'].T).reshape(B, L, -1)
    Di = xz.shape[-1] // 2
    xin, z = xz[..., :Di], xz[..., Di:]
    kw = p['conv_w'].shape[-1]
    xc = jax.lax.conv_general_dilated(xin, p['conv_w'].transpose(2, 1, 0), (1,), [(kw - 1, 0)],
                                      feature_group_count=Di, dimension_numbers=('NWC', 'WIO', 'NWC'))
    xc = jax.nn.silu(xc + p['conv_b'])
    R = p['dt_proj_w'].shape[1]
    proj = xc @ p['x_proj_w'].T
    dt = jax.nn.softplus(proj[..., :R] @ p['dt_proj_w'].T + p['dt_proj_b'])
    Bm, Cm = proj[..., R:R + N], proj[..., R + N:]
    A = -jnp.exp(p['A_log'])
    dA = jnp.exp(dt[..., None] * A)
    dBx = (dt * xc)[..., None] * Bm[:, :, None, :]
    h = jnp.zeros((B, Di, N), x.dtype)
    ys = []
    for t in range(L):
        h = dA[:, t] * h + dBx[:, t]
        ys.append(jnp.einsum('bdn,bn->bd', h, Cm[:, t]))
    y = jnp.stack(ys, axis=1) + xc * p['D']
    g = (y * jax.nn.silu(z)).reshape(B * L, Di)
    return _mm(g, p['out_proj_w'].T).reshape(B, L, D)


def _spa_branch(x, p):
    xp = _lsp(x, p['lsp'])
    xr = _prca(xp, p['prca'])
    B, H, W, C = xr.shape
    xf = _mamba_long(xr.reshape(B, H * W, C), p['mamba'])
    xr = xf.reshape(B, H, W, C)
    return jax.nn.silu(_gn(xr, p['gn_w'], p['gn_b'], 4)) + xp


def _spe_branch(x, p, token_num=4):
    xr = _prca(x, p['prca'])
    B, H, W, C = xr.shape
    gc = C // token_num
    xf = _mamba_short(xr.reshape(B * H * W, token_num, gc), p['mamba'])
    xr2 = xf.reshape(B, H, W, C)
    return jax.nn.silu(_gn(xr2, p['gn_w'], p['gn_b'], 4)) + x


def _gap(x):
    return x.mean((1, 2))


def _gate(x, w1, w2):
    return jax.nn.sigmoid(jax.nn.silu(_gap(x) @ w1.T) @ w2.T)[:, None, None, :]


def _bridge(spa, spe, p):
    spa_out = spa + p['gamma_spa'] * (_gate(spe, p['spa_w1'], p['spa_w2']) * spa)
    spe_out = spe + p['gamma_spe'] * (_gate(spa, p['spe_w1'], p['spe_w2']) * spe)
    return spa_out, spe_out


def _ccaf(spa, spe, p):
    logit = jnp.stack([_gap(spa) @ p['fc_spa'].T, _gap(spe) @ p['fc_spe'].T], axis=1)
    w = jax.nn.softmax(logit, axis=1)[:, :, None, None, :]
    g_cons = _gate(spa * spe, p['cons_w1'], p['cons_w2'])
    g_conf = _gate(jnp.abs(spa - spe), p['conf_w1'], p['conf_w2'])
    competitive = w[:, 0] * spa + w[:, 1] * spe
    consensus = g_cons * 0.5 * (spa + spe)
    return competitive + p['beta'] * consensus * (1.0 - g_conf)


def kernel(x, params):
    xh = x.transpose(0, 2, 3, 1)
    spa = _spa_branch(xh, params['spa'])
    spe = _spe_branch(xh, params['spe'])
    spa, spe = _bridge(spa, spe, params['bridge'])
    out = _ccaf(spa, spe, params['ccaf'])
    return out.transpose(0, 3, 1, 2)


# speedup vs baseline: 1.0494x; 1.0494x over previous
import functools
import math

import jax
import jax.numpy as jnp
from jax.experimental import pallas as pl
from jax.experimental.pallas import tpu as pltpu

_PAD = 128


def _mm_body(act, x_ref, w_ref, b_ref, o_ref):
    y = jnp.dot(x_ref[...], w_ref[...], preferred_element_type=jnp.float32) + b_ref[...]
    if act == 'silu':
        y = y * jax.nn.sigmoid(y)
    o_ref[...] = y


def _mm(x, w, b=None, act=None, bm=512):
    M, K = x.shape
    N = w.shape[1]
    b2 = jnp.zeros((1, N), jnp.float32) if b is None else b.reshape(1, N)
    return pl.pallas_call(
        functools.partial(_mm_body, act),
        out_shape=jax.ShapeDtypeStruct((M, N), jnp.float32),
        grid=(M // bm,),
        in_specs=[pl.BlockSpec((bm, K), lambda i: (i, 0)),
                  pl.BlockSpec((K, N), lambda i: (0, 0)),
                  pl.BlockSpec((1, N), lambda i: (0, 0))],
        out_specs=pl.BlockSpec((bm, N), lambda i: (i, 0)),
        compiler_params=pltpu.CompilerParams(
            dimension_semantics=("arbitrary",),
            vmem_limit_bytes=100 * 1024 * 1024,
        ),
        name=f"mm_{N}",
    )(x, w, b2)


def _attn_chain_body(Wd, nlayers, x_ref, wqkv_ref, bqkv_ref, wdw_ref, bdw_ref,
                     temp_ref, wproj_ref, bproj_ref, o_ref, qpad_ref):
    n, C = x_ref.shape[1], x_ref.shape[2]
    C3 = 3 * C
    hd = C // 4

    qpad_ref[:_PAD] = jnp.zeros((_PAD, C3), jnp.float32)
    qpad_ref[_PAD + n:] = jnp.zeros((_PAD, C3), jnp.float32)

    rx = jax.lax.broadcasted_iota(jnp.int32, (n // Wd, Wd, C3), 1).reshape(n, C3)
    mp = jnp.where(rx < Wd - 2, 1.0, 0.0)
    mm_ = jnp.where(rx >= 2, 1.0, 0.0)

    y = x_ref[0]
    for l in range(nlayers):
        qkv = jax.lax.dot_general(y, wqkv_ref[l], (((1,), (1,)), ((), ())),
                                  preferred_element_type=jnp.float32) + bqkv_ref[l]
        qpad_ref[_PAD:_PAD + n] = qkv
        wd = wdw_ref[l]
        acc0 = jnp.zeros((n, C3), jnp.float32)
        accm = jnp.zeros((n, C3), jnp.float32)
        accp = jnp.zeros((n, C3), jnp.float32)
        for ky in range(3):
            off = (ky - 1) * 2 * Wd
            accm = accm + wd[3 * ky:3 * ky + 1] * qpad_ref[_PAD + off - 2:_PAD + off - 2 + n]
            acc0 = acc0 + wd[3 * ky + 1:3 * ky + 2] * qpad_ref[_PAD + off:_PAD + off + n]
            accp = accp + wd[3 * ky + 2:3 * ky + 3] * qpad_ref[_PAD + off + 2:_PAD + off + 2 + n]
        conv = acc0 + mm_ * accm + mp * accp + bdw_ref[l]

        outs = []
        for h in range(4):
            qh = conv[:, h * hd:(h + 1) * hd]
            kh = conv[:, C + h * hd:C + (h + 1) * hd]
            vh = conv[:, 2 * C + h * hd:2 * C + (h + 1) * hd]
            qn = qh * jax.lax.rsqrt(jnp.maximum(jnp.sum(qh * qh, axis=0, keepdims=True), 1e-24))
            kn = kh * jax.lax.rsqrt(jnp.maximum(jnp.sum(kh * kh, axis=0, keepdims=True), 1e-24))
            att = jax.lax.dot_general(qn, kn, (((0,), (0,)), ((), ())),
                                      preferred_element_type=jnp.float32)
            att = att * temp_ref[l, h]
            att = att - jnp.max(att, axis=-1, keepdims=True)
            att = jnp.exp(att)
            att = att / jnp.sum(att, axis=-1, keepdims=True)
            outs.append(jax.lax.dot_general(vh, att, (((1,), (1,)), ((), ())),
                                            preferred_element_type=jnp.float32))
        hall = jnp.concatenate(outs, axis=-1)
        y = jax.lax.dot_general(hall, wproj_ref[l], (((1,), (1,)), ((), ())),
                                preferred_element_type=jnp.float32) + bproj_ref[l]
    o_ref[0] = y


def _attn_chain(x, aps, Wd):
    B, n, C = x.shape
    C3 = 3 * C
    L = len(aps)
    wqkv = jnp.stack([ap['qkv_w'][:, :, 0, 0] for ap in aps])
    bqkv = jnp.stack([ap['qkv_b'][None, :] for ap in aps])
    wdw = jnp.stack([ap['dw_w'][:, 0].reshape(C3, 9).T for ap in aps])
    bdw = jnp.stack([ap['dw_b'][None, :] for ap in aps])
    temp = jnp.stack([ap['temp'][:, 0, 0] for ap in aps])
    wproj = jnp.stack([ap['proj_w'][:, :, 0, 0] for ap in aps])
    bproj = jnp.stack([ap['proj_b'][None, :] for ap in aps])

    body = functools.partial(_attn_chain_body, Wd, L)
    return pl.pallas_call(
        body,
        out_shape=jax.ShapeDtypeStruct((B, n, C), jnp.float32),
        grid=(B,),
        in_specs=[
            pl.BlockSpec((1, n, C), lambda b: (b, 0, 0)),
            pl.BlockSpec((L, C3, C), lambda b: (0, 0, 0)),
            pl.BlockSpec((L, 1, C3), lambda b: (0, 0, 0)),
            pl.BlockSpec((L, 9, C3), lambda b: (0, 0, 0)),
            pl.BlockSpec((L, 1, C3), lambda b: (0, 0, 0)),
            pl.BlockSpec((L, 4), lambda b: (0, 0), memory_space=pltpu.SMEM),
            pl.BlockSpec((L, C, C), lambda b: (0, 0, 0)),
            pl.BlockSpec((L, 1, C), lambda b: (0, 0, 0)),
        ],
        out_specs=pl.BlockSpec((1, n, C), lambda b: (b, 0, 0)),
        scratch_shapes=[pltpu.VMEM((n + 2 * _PAD, C3), jnp.float32)],
        compiler_params=pltpu.CompilerParams(
            dimension_semantics=("arbitrary",),
            vmem_limit_bytes=110 * 1024 * 1024,
        ),
        name=f"attn_chain_{Wd}",
    )(x, wqkv, bqkv, wdw, bdw, temp, wproj, bproj)


def _gn(x, w, b, groups, eps=1e-5):
    B, H, W, C = x.shape
    xg = x.reshape(B, H * W, groups, C // groups)
    m = xg.mean((1, 3), keepdims=True)
    v = ((xg - m) ** 2).mean((1, 3), keepdims=True)
    xg = (xg - m) * jax.lax.rsqrt(v + eps)
    return xg.reshape(B, H, W, C) * w + b


def _lsp(x, p):
    B, H, W, C = x.shape
    local = jax.lax.conv_general_dilated(
        x, p['dw_w'].transpose(2, 3, 1, 0), (1, 1), [(1, 1), (1, 1)],
        feature_group_count=C, dimension_numbers=('NHWC', 'HWIO', 'NHWC'))
    local = local + p['dw_b']
    x2d = x.reshape(B * H * W, C)
    g1 = _mm(x2d, p['g1_w'][:, :, 0, 0].T, p['g1_b'], act='silu')
    gate = jax.nn.sigmoid(g1 @ p['g2_w'][:, :, 0, 0].T + p['g2_b'])
    lg = local.reshape(B * H * W, C) * gate
    out = _mm(lg, p['pw_w'][:, :, 0, 0].T, p['pw_b']).reshape(B, H, W, C)
    return jax.nn.silu(_gn(out, p['gn_w'], p['gn_b'], 4)) + x


def _avgpool(x, k):
    B, H, W, C = x.shape
    return x.reshape(B, H // k, k, W // k, k, C).mean((2, 4))


def _prca(x, p):
    B, H, W, C = x.shape
    outs = []
    for i in range(len(p['attn'])):
        o = x if i == 0 else _avgpool(x, 2 ** i)
        Wd = W // (2 ** i)
        o3 = _attn_chain(o.reshape(B, Wd * Wd, C), p['attn'][i], Wd)
        o = o3.reshape(B, Wd, Wd, C)
        if i > 0:
            o = jax.image.resize(o, (B, H, W, C), 'bilinear')
        outs.append(o)
    cat = jnp.concatenate([o.reshape(B * H * W, C) for o in outs], axis=-1)
    out = _mm(cat, p['proj_w'][:, :, 0, 0].T, p['proj_b'])
    return out.reshape(B, H, W, C)


_Q = 64
_NST = 16


def _scan_body(dt_ref, dtx_ref, xc_ref, z_ref, bmd_ref, cmd_ref,
               rep_ref, arep_ref, dvec_ref, ow_ref, o_ref, hc_ref, H_ref):
    c = pl.program_id(1)

    @pl.when(c == 0)
    def _():
        hc_ref[...] = jnp.zeros_like(hc_ref)

    dtc = dt_ref[0]
    dt_bc = jnp.dot(rep_ref[...], dtc, preferred_element_type=jnp.float32)
    a = jnp.exp(dt_bc * arep_ref[...])
    bmat = jnp.dot(bmd_ref[0, 0], dtx_ref[0], preferred_element_type=jnp.float32)

    h = hc_ref[...]
    for s in range(_Q):
        h = a[s * _NST:(s + 1) * _NST] * h + bmat[s * _NST:(s + 1) * _NST]
        H_ref[s * _NST:(s + 1) * _NST, :] = h
    hc_ref[...] = h

    y = jnp.dot(cmd_ref[0, 0], H_ref[...], preferred_element_type=jnp.float32)
    y = y + xc_ref[0] * dvec_ref[...]
    z = z_ref[0]
    y = y * (z * jax.nn.sigmoid(z))
    o_ref[0] = jnp.dot(y, ow_ref[...], preferred_element_type=jnp.float32)


def _mamba_scan_pallas(dt, xc, z, Bm, Cm, A, Dvec, out_w):
    B, L, Di = dt.shape
    N = Bm.shape[-1]
    Q = _Q
    NC = L // Q
    QN = Q * N

    eyeQ = jnp.eye(Q, dtype=jnp.float32)
    Bm_r = Bm.reshape(B, NC, Q, N)
    Cm_r = Cm.reshape(B, NC, Q, N)
    bmdiag = (Bm_r[:, :, :, :, None] * eyeQ[None, None, :, None, :]).reshape(B, NC, QN, Q)
    cmdiag = (Cm_r[:, :, :, None, :] * eyeQ[None, None, :, :, None]).reshape(B, NC, Q, QN)
    rep = jnp.repeat(eyeQ, N, axis=0)
    arep = jnp.tile(A.T, (Q, 1))
    dtx = dt * xc
    dvec = Dvec[None, :]
    owT = out_w.T
    D = out_w.shape[0]

    grid = (B, NC)
    seq3 = pl.BlockSpec((1, Q, Di), lambda b, c: (b, c, 0))
    out = pl.pallas_call(
        _scan_body,
        out_shape=jax.ShapeDtypeStruct((B, L, D), jnp.float32),
        grid=grid,
        in_specs=[
            seq3,
            seq3,
            seq3,
            seq3,
            pl.BlockSpec((1, 1, QN, Q), lambda b, c: (b, c, 0, 0)),
            pl.BlockSpec((1, 1, Q, QN), lambda b, c: (b, c, 0, 0)),
            pl.BlockSpec((QN, Q), lambda b, c: (0, 0)),
            pl.BlockSpec((QN, Di), lambda b, c: (0, 0)),
            pl.BlockSpec((1, Di), lambda b, c: (0, 0)),
            pl.BlockSpec((Di, D), lambda b, c: (0, 0)),
        ],
        out_specs=pl.BlockSpec((1, Q, D), lambda b, c: (b, c, 0)),
        scratch_shapes=[
            pltpu.VMEM((N, Di), jnp.float32),
            pltpu.VMEM((QN, Di), jnp.float32),
        ],
        compiler_params=pltpu.CompilerParams(
            dimension_semantics=("arbitrary", "arbitrary"),
            vmem_limit_bytes=100 * 1024 * 1024,
        ),
        name="mamba_scan",
    )(dt, dtx, xc, z, bmdiag, cmdiag, rep, arep, dvec, owT)
    return out


def _mamba_long(x, p, N=16):
    B, L, D = x.shape
    xz = _mm(x.reshape(B * L, D), p['in_proj_w'].T).reshape(B, L, -1)
    Di = xz.shape[-1] // 2
    xin, z = xz[..., :Di], xz[..., Di:]
    kw = p['conv_w'].shape[-1]
    xc = jax.lax.conv_general_dilated(xin, p['conv_w'].transpose(2, 1, 0), (1,), [(kw - 1, 0)],
                                      feature_group_count=Di, dimension_numbers=('NWC', 'WIO', 'NWC'))
    xc = jax.nn.silu(xc + p['conv_b'])
    R = p['dt_proj_w'].shape[1]
    proj = xc @ p['x_proj_w'].T
    dt = jax.nn.softplus(proj[..., :R] @ p['dt_proj_w'].T + p['dt_proj_b'])
    Bm, Cm = proj[..., R:R + N], proj[..., R + N:]
    A = -jnp.exp(p['A_log'])
    return _mamba_scan_pallas(dt, xc, z, Bm, Cm, A, p['D'], p['out_proj_w'])


def _spe_scan_body(T, Di, N, dt_ref, dtx_ref, xc_ref, z_ref, bmc_ref, cmc_ref,
                   ebt_ref, af_ref, dv_ref, o_ref):
    SB = dt_ref.shape[0]
    DN = Di * N
    h = jnp.zeros((SB, DN), jnp.float32)
    for t in range(T):
        bex = jnp.dot(bmc_ref[:, t * N:(t + 1) * N], ebt_ref[...],
                      preferred_element_type=jnp.float32)
        cex = jnp.dot(cmc_ref[:, t * N:(t + 1) * N], ebt_ref[...],
                      preferred_element_type=jnp.float32)
        dt_t = dt_ref[:, t * Di:(t + 1) * Di]
        dtt = jnp.tile(jnp.concatenate([dt_t, dt_t], axis=1), (1, DN // (2 * Di)))
        a = jnp.exp(dtt * af_ref[...])
        dtx_t = dtx_ref[:, t * Di:(t + 1) * Di]
        dxt = jnp.tile(jnp.concatenate([dtx_t, dtx_t], axis=1), (1, DN // (2 * Di)))
        h = a * h + bex * dxt
        r = cex * h
        w = DN
        while w > Di:
            w //= 2
            r = r[:, :w] + r[:, w:2 * w]
        y = r + xc_ref[:, t * Di:(t + 1) * Di] * dv_ref[...]
        zt = z_ref[:, t * Di:(t + 1) * Di]
        o_ref[:, t * Di:(t + 1) * Di] = y * (zt * jax.nn.sigmoid(zt))


def _mamba_short(x, p, N=16):
    S, T, D = x.shape
    xz = _mm(x.reshape(S * T, D), p['in_proj_w'].T).reshape(S, T, -1)
    Di = xz.shape[-1] // 2
    xin, z = xz[..., :Di], xz[..., Di:]
    kw = p['conv_w'].shape[-1]
    xc = jax.lax.conv_general_dilated(xin, p['conv_w'].transpose(2, 1, 0), (1,), [(kw - 1, 0)],
                                      feature_group_count=Di, dimension_numbers=('NWC', 'WIO', 'NWC'))
    xc = jax.nn.silu(xc + p['conv_b'])
    R = p['dt_proj_w'].shape[1]
    proj = xc @ p['x_proj_w'].T
    dt = jax.nn.softplus(proj[..., :R] @ p['dt_proj_w'].T + p['dt_proj_b'])
    Bm, Cm = proj[..., R:R + N], proj[..., R + N:]
    A = -jnp.exp(p['A_log'])

    DN = Di * N
    af = A.T.reshape(1, DN)
    ebt = jnp.repeat(jnp.eye(N, dtype=jnp.float32), Di, axis=1)
    dv = p['D'].reshape(1, Di)
    dtf = dt.reshape(S, T * Di)
    dtxf = (dt * xc).reshape(S, T * Di)
    xcf = xc.reshape(S, T * Di)
    zf = z.reshape(S, T * Di)
    bmf = Bm.reshape(S, T * N)
    cmf = Cm.reshape(S, T * N)

    SB = 128
    body = functools.partial(_spe_scan_body, T, Di, N)
    seq = pl.BlockSpec((SB, T * Di), lambda i: (i, 0))
    g = pl.pallas_call(
        body,
        out_shape=jax.ShapeDtypeStruct((S, T * Di), jnp.float32),
        grid=(S // SB,),
        in_specs=[
            seq, seq, seq, seq,
            pl.BlockSpec((SB, T * N), lambda i: (i, 0)),
            pl.BlockSpec((SB, T * N), lambda i: (i, 0)),
            pl.BlockSpec((N, DN), lambda i: (0, 0)),
            pl.BlockSpec((1, DN), lambda i: (0, 0)),
            pl.BlockSpec((1, Di), lambda i: (0, 0)),
        ],
        out_specs=seq,
        compiler_params=pltpu.CompilerParams(
            dimension_semantics=("arbitrary",),
            vmem_limit_bytes=100 * 1024 * 1024,
        ),
        name="spe_scan",
    )(dtf, dtxf, xcf, zf, bmf, cmf, ebt, af, dv)
    return _mm(g.reshape(S * T, Di), p['out_proj_w'].T).reshape(S, T, D)


def _spa_branch(x, p):
    xp = _lsp(x, p['lsp'])
    xr = _prca(xp, p['prca'])
    B, H, W, C = xr.shape
    xf = _mamba_long(xr.reshape(B, H * W, C), p['mamba'])
    xr = xf.reshape(B, H, W, C)
    return jax.nn.silu(_gn(xr, p['gn_w'], p['gn_b'], 4)) + xp


def _spe_branch(x, p, token_num=4):
    xr = _prca(x, p['prca'])
    B, H, W, C = xr.shape
    gc = C // token_num
    xf = _mamba_short(xr.reshape(B * H * W, token_num, gc), p['mamba'])
    xr2 = xf.reshape(B, H, W, C)
    return jax.nn.silu(_gn(xr2, p['gn_w'], p['gn_b'], 4)) + x


def _gap(x):
    return x.mean((1, 2))


def _gate(x, w1, w2):
    return jax.nn.sigmoid(jax.nn.silu(_gap(x) @ w1.T) @ w2.T)[:, None, None, :]


def _bridge(spa, spe, p):
    spa_out = spa + p['gamma_spa'] * (_gate(spe, p['spa_w1'], p['spa_w2']) * spa)
    spe_out = spe + p['gamma_spe'] * (_gate(spa, p['spe_w1'], p['spe_w2']) * spe)
    return spa_out, spe_out


def _ccaf(spa, spe, p):
    logit = jnp.stack([_gap(spa) @ p['fc_spa'].T, _gap(spe) @ p['fc_spe'].T], axis=1)
    w = jax.nn.softmax(logit, axis=1)[:, :, None, None, :]
    g_cons = _gate(spa * spe, p['cons_w1'], p['cons_w2'])
    g_conf = _gate(jnp.abs(spa - spe), p['conf_w1'], p['conf_w2'])
    competitive = w[:, 0] * spa + w[:, 1] * spe
    consensus = g_cons * 0.5 * (spa + spe)
    return competitive + p['beta'] * consensus * (1.0 - g_conf)


def kernel(x, params):
    xh = x.transpose(0, 2, 3, 1)
    spa = _spa_branch(xh, params['spa'])
    spe = _spe_branch(xh, params['spe'])
    spa, spe = _bridge(spa, spe, params['bridge'])
    out = _ccaf(spa, spe, params['ccaf'])
    return out.transpose(0, 3, 1, 2)


# speedup vs baseline: 1.0494x; 1.0000x over previous
import functools
import math

import jax
import jax.numpy as jnp
from jax.experimental import pallas as pl
from jax.experimental.pallas import tpu as pltpu

_PAD = 128


def _mm_body(act, x_ref, w_ref, b_ref, o_ref):
    y = jnp.dot(x_ref[...], w_ref[...], preferred_element_type=jnp.float32) + b_ref[...]
    if act == 'silu':
        y = y * jax.nn.sigmoid(y)
    o_ref[...] = y


def _mm(x, w, b=None, act=None, bm=512):
    M, K = x.shape
    N = w.shape[1]
    b2 = jnp.zeros((1, N), jnp.float32) if b is None else b.reshape(1, N)
    return pl.pallas_call(
        functools.partial(_mm_body, act),
        out_shape=jax.ShapeDtypeStruct((M, N), jnp.float32),
        grid=(M // bm,),
        in_specs=[pl.BlockSpec((bm, K), lambda i: (i, 0)),
                  pl.BlockSpec((K, N), lambda i: (0, 0)),
                  pl.BlockSpec((1, N), lambda i: (0, 0))],
        out_specs=pl.BlockSpec((bm, N), lambda i: (i, 0)),
        compiler_params=pltpu.CompilerParams(
            dimension_semantics=("parallel",),
            vmem_limit_bytes=100 * 1024 * 1024,
        ),
        name=f"mm_{N}",
    )(x, w, b2)


def _attn_chain_body(Wd, nlayers, x_ref, wqkv_ref, bqkv_ref, wdw_ref, bdw_ref,
                     temp_ref, wproj_ref, bproj_ref, o_ref, qpad_ref):
    n, C = x_ref.shape[1], x_ref.shape[2]
    C3 = 3 * C
    hd = C // 4

    qpad_ref[:_PAD] = jnp.zeros((_PAD, C3), jnp.float32)
    qpad_ref[_PAD + n:] = jnp.zeros((_PAD, C3), jnp.float32)

    rx = jax.lax.broadcasted_iota(jnp.int32, (n // Wd, Wd, C3), 1).reshape(n, C3)
    mp = jnp.where(rx < Wd - 2, 1.0, 0.0)
    mm_ = jnp.where(rx >= 2, 1.0, 0.0)

    y = x_ref[0]
    for l in range(nlayers):
        qkv = jax.lax.dot_general(y, wqkv_ref[l], (((1,), (1,)), ((), ())),
                                  preferred_element_type=jnp.float32) + bqkv_ref[l]
        qpad_ref[_PAD:_PAD + n] = qkv
        wd = wdw_ref[l]
        acc0 = jnp.zeros((n, C3), jnp.float32)
        accm = jnp.zeros((n, C3), jnp.float32)
        accp = jnp.zeros((n, C3), jnp.float32)
        for ky in range(3):
            off = (ky - 1) * 2 * Wd
            accm = accm + wd[3 * ky:3 * ky + 1] * qpad_ref[_PAD + off - 2:_PAD + off - 2 + n]
            acc0 = acc0 + wd[3 * ky + 1:3 * ky + 2] * qpad_ref[_PAD + off:_PAD + off + n]
            accp = accp + wd[3 * ky + 2:3 * ky + 3] * qpad_ref[_PAD + off + 2:_PAD + off + 2 + n]
        conv = acc0 + mm_ * accm + mp * accp + bdw_ref[l]

        outs = []
        for h in range(4):
            qh = conv[:, h * hd:(h + 1) * hd]
            kh = conv[:, C + h * hd:C + (h + 1) * hd]
            vh = conv[:, 2 * C + h * hd:2 * C + (h + 1) * hd]
            qn = qh * jax.lax.rsqrt(jnp.maximum(jnp.sum(qh * qh, axis=0, keepdims=True), 1e-24))
            kn = kh * jax.lax.rsqrt(jnp.maximum(jnp.sum(kh * kh, axis=0, keepdims=True), 1e-24))
            att = jax.lax.dot_general(qn, kn, (((0,), (0,)), ((), ())),
                                      preferred_element_type=jnp.float32)
            att = att * temp_ref[l, h]
            att = att - jnp.max(att, axis=-1, keepdims=True)
            att = jnp.exp(att)
            att = att / jnp.sum(att, axis=-1, keepdims=True)
            outs.append(jax.lax.dot_general(vh, att, (((1,), (1,)), ((), ())),
                                            preferred_element_type=jnp.float32))
        hall = jnp.concatenate(outs, axis=-1)
        y = jax.lax.dot_general(hall, wproj_ref[l], (((1,), (1,)), ((), ())),
                                preferred_element_type=jnp.float32) + bproj_ref[l]
    o_ref[0] = y


def _attn_chain(x, aps, Wd):
    B, n, C = x.shape
    C3 = 3 * C
    L = len(aps)
    wqkv = jnp.stack([ap['qkv_w'][:, :, 0, 0] for ap in aps])
    bqkv = jnp.stack([ap['qkv_b'][None, :] for ap in aps])
    wdw = jnp.stack([ap['dw_w'][:, 0].reshape(C3, 9).T for ap in aps])
    bdw = jnp.stack([ap['dw_b'][None, :] for ap in aps])
    temp = jnp.stack([ap['temp'][:, 0, 0] for ap in aps])
    wproj = jnp.stack([ap['proj_w'][:, :, 0, 0] for ap in aps])
    bproj = jnp.stack([ap['proj_b'][None, :] for ap in aps])

    body = functools.partial(_attn_chain_body, Wd, L)
    return pl.pallas_call(
        body,
        out_shape=jax.ShapeDtypeStruct((B, n, C), jnp.float32),
        grid=(B,),
        in_specs=[
            pl.BlockSpec((1, n, C), lambda b: (b, 0, 0)),
            pl.BlockSpec((L, C3, C), lambda b: (0, 0, 0)),
            pl.BlockSpec((L, 1, C3), lambda b: (0, 0, 0)),
            pl.BlockSpec((L, 9, C3), lambda b: (0, 0, 0)),
            pl.BlockSpec((L, 1, C3), lambda b: (0, 0, 0)),
            pl.BlockSpec((L, 4), lambda b: (0, 0), memory_space=pltpu.SMEM),
            pl.BlockSpec((L, C, C), lambda b: (0, 0, 0)),
            pl.BlockSpec((L, 1, C), lambda b: (0, 0, 0)),
        ],
        out_specs=pl.BlockSpec((1, n, C), lambda b: (b, 0, 0)),
        scratch_shapes=[pltpu.VMEM((n + 2 * _PAD, C3), jnp.float32)],
        compiler_params=pltpu.CompilerParams(
            dimension_semantics=("parallel",),
            vmem_limit_bytes=110 * 1024 * 1024,
        ),
        name=f"attn_chain_{Wd}",
    )(x, wqkv, bqkv, wdw, bdw, temp, wproj, bproj)


def _gn(x, w, b, groups, eps=1e-5):
    B, H, W, C = x.shape
    xg = x.reshape(B, H * W, groups, C // groups)
    m = xg.mean((1, 3), keepdims=True)
    v = ((xg - m) ** 2).mean((1, 3), keepdims=True)
    xg = (xg - m) * jax.lax.rsqrt(v + eps)
    return xg.reshape(B, H, W, C) * w + b


def _lsp(x, p):
    B, H, W, C = x.shape
    local = jax.lax.conv_general_dilated(
        x, p['dw_w'].transpose(2, 3, 1, 0), (1, 1), [(1, 1), (1, 1)],
        feature_group_count=C, dimension_numbers=('NHWC', 'HWIO', 'NHWC'))
    local = local + p['dw_b']
    x2d = x.reshape(B * H * W, C)
    g1 = _mm(x2d, p['g1_w'][:, :, 0, 0].T, p['g1_b'], act='silu')
    gate = jax.nn.sigmoid(g1 @ p['g2_w'][:, :, 0, 0].T + p['g2_b'])
    lg = local.reshape(B * H * W, C) * gate
    out = _mm(lg, p['pw_w'][:, :, 0, 0].T, p['pw_b']).reshape(B, H, W, C)
    return jax.nn.silu(_gn(out, p['gn_w'], p['gn_b'], 4)) + x


def _avgpool(x, k):
    B, H, W, C = x.shape
    return x.reshape(B, H // k, k, W // k, k, C).mean((2, 4))


def _prca(x, p):
    B, H, W, C = x.shape
    outs = []
    for i in range(len(p['attn'])):
        o = x if i == 0 else _avgpool(x, 2 ** i)
        Wd = W // (2 ** i)
        o3 = _attn_chain(o.reshape(B, Wd * Wd, C), p['attn'][i], Wd)
        o = o3.reshape(B, Wd, Wd, C)
        if i > 0:
            o = jax.image.resize(o, (B, H, W, C), 'bilinear')
        outs.append(o)
    cat = jnp.concatenate([o.reshape(B * H * W, C) for o in outs], axis=-1)
    out = _mm(cat, p['proj_w'][:, :, 0, 0].T, p['proj_b'])
    return out.reshape(B, H, W, C)


_Q = 64
_NST = 16


def _scan_body(dt_ref, dtx_ref, xc_ref, z_ref, bmd_ref, cmd_ref,
               rep_ref, arep_ref, dvec_ref, ow_ref, o_ref, hc_ref, H_ref):
    c = pl.program_id(1)

    @pl.when(c == 0)
    def _():
        hc_ref[...] = jnp.zeros_like(hc_ref)

    dtc = dt_ref[0]
    dt_bc = jnp.dot(rep_ref[...], dtc, preferred_element_type=jnp.float32)
    a = jnp.exp(dt_bc * arep_ref[...])
    bmat = jnp.dot(bmd_ref[0, 0], dtx_ref[0], preferred_element_type=jnp.float32)

    h = hc_ref[...]
    for s in range(_Q):
        h = a[s * _NST:(s + 1) * _NST] * h + bmat[s * _NST:(s + 1) * _NST]
        H_ref[s * _NST:(s + 1) * _NST, :] = h
    hc_ref[...] = h

    y = jnp.dot(cmd_ref[0, 0], H_ref[...], preferred_element_type=jnp.float32)
    y = y + xc_ref[0] * dvec_ref[...]
    z = z_ref[0]
    y = y * (z * jax.nn.sigmoid(z))
    o_ref[0] = jnp.dot(y, ow_ref[...], preferred_element_type=jnp.float32)


def _mamba_scan_pallas(dt, xc, z, Bm, Cm, A, Dvec, out_w):
    B, L, Di = dt.shape
    N = Bm.shape[-1]
    Q = _Q
    NC = L // Q
    QN = Q * N

    eyeQ = jnp.eye(Q, dtype=jnp.float32)
    Bm_r = Bm.reshape(B, NC, Q, N)
    Cm_r = Cm.reshape(B, NC, Q, N)
    bmdiag = (Bm_r[:, :, :, :, None] * eyeQ[None, None, :, None, :]).reshape(B, NC, QN, Q)
    cmdiag = (Cm_r[:, :, :, None, :] * eyeQ[None, None, :, :, None]).reshape(B, NC, Q, QN)
    rep = jnp.repeat(eyeQ, N, axis=0)
    arep = jnp.tile(A.T, (Q, 1))
    dtx = dt * xc
    dvec = Dvec[None, :]
    owT = out_w.T
    D = out_w.shape[0]

    grid = (B, NC)
    seq3 = pl.BlockSpec((1, Q, Di), lambda b, c: (b, c, 0))
    out = pl.pallas_call(
        _scan_body,
        out_shape=jax.ShapeDtypeStruct((B, L, D), jnp.float32),
        grid=grid,
        in_specs=[
            seq3,
            seq3,
            seq3,
            seq3,
            pl.BlockSpec((1, 1, QN, Q), lambda b, c: (b, c, 0, 0)),
            pl.BlockSpec((1, 1, Q, QN), lambda b, c: (b, c, 0, 0)),
            pl.BlockSpec((QN, Q), lambda b, c: (0, 0)),
            pl.BlockSpec((QN, Di), lambda b, c: (0, 0)),
            pl.BlockSpec((1, Di), lambda b, c: (0, 0)),
            pl.BlockSpec((Di, D), lambda b, c: (0, 0)),
        ],
        out_specs=pl.BlockSpec((1, Q, D), lambda b, c: (b, c, 0)),
        scratch_shapes=[
            pltpu.VMEM((N, Di), jnp.float32),
            pltpu.VMEM((QN, Di), jnp.float32),
        ],
        compiler_params=pltpu.CompilerParams(
            dimension_semantics=("parallel", "arbitrary"),
            vmem_limit_bytes=100 * 1024 * 1024,
        ),
        name="mamba_scan",
    )(dt, dtx, xc, z, bmdiag, cmdiag, rep, arep, dvec, owT)
    return out


def _mamba_long(x, p, N=16):
    B, L, D = x.shape
    xz = _mm(x.reshape(B * L, D), p['in_proj_w'].T).reshape(B, L, -1)
    Di = xz.shape[-1] // 2
    xin, z = xz[..., :Di], xz[..., Di:]
    kw = p['conv_w'].shape[-1]
    xc = jax.lax.conv_general_dilated(xin, p['conv_w'].transpose(2, 1, 0), (1,), [(kw - 1, 0)],
                                      feature_group_count=Di, dimension_numbers=('NWC', 'WIO', 'NWC'))
    xc = jax.nn.silu(xc + p['conv_b'])
    R = p['dt_proj_w'].shape[1]
    proj = xc @ p['x_proj_w'].T
    dt = jax.nn.softplus(proj[..., :R] @ p['dt_proj_w'].T + p['dt_proj_b'])
    Bm, Cm = proj[..., R:R + N], proj[..., R + N:]
    A = -jnp.exp(p['A_log'])
    return _mamba_scan_pallas(dt, xc, z, Bm, Cm, A, p['D'], p['out_proj_w'])


def _spe_scan_body(T, Di, N, dt_ref, dtx_ref, xc_ref, z_ref, bmc_ref, cmc_ref,
                   ebt_ref, af_ref, dv_ref, o_ref):
    SB = dt_ref.shape[0]
    DN = Di * N
    h = jnp.zeros((SB, DN), jnp.float32)
    for t in range(T):
        bex = jnp.dot(bmc_ref[:, t * N:(t + 1) * N], ebt_ref[...],
                      preferred_element_type=jnp.float32)
        cex = jnp.dot(cmc_ref[:, t * N:(t + 1) * N], ebt_ref[...],
                      preferred_element_type=jnp.float32)
        dt_t = dt_ref[:, t * Di:(t + 1) * Di]
        dtt = jnp.tile(jnp.concatenate([dt_t, dt_t], axis=1), (1, DN // (2 * Di)))
        a = jnp.exp(dtt * af_ref[...])
        dtx_t = dtx_ref[:, t * Di:(t + 1) * Di]
        dxt = jnp.tile(jnp.concatenate([dtx_t, dtx_t], axis=1), (1, DN // (2 * Di)))
        h = a * h + bex * dxt
        r = cex * h
        w = DN
        while w > Di:
            w //= 2
            r = r[:, :w] + r[:, w:2 * w]
        y = r + xc_ref[:, t * Di:(t + 1) * Di] * dv_ref[...]
        zt = z_ref[:, t * Di:(t + 1) * Di]
        o_ref[:, t * Di:(t + 1) * Di] = y * (zt * jax.nn.sigmoid(zt))


def _mamba_short(x, p, N=16):
    S, T, D = x.shape
    xz = _mm(x.reshape(S * T, D), p['in_proj_w'].T).reshape(S, T, -1)
    Di = xz.shape[-1] // 2
    xin, z = xz[..., :Di], xz[..., Di:]
    kw = p['conv_w'].shape[-1]
    xc = jax.lax.conv_general_dilated(xin, p['conv_w'].transpose(2, 1, 0), (1,), [(kw - 1, 0)],
                                      feature_group_count=Di, dimension_numbers=('NWC', 'WIO', 'NWC'))
    xc = jax.nn.silu(xc + p['conv_b'])
    R = p['dt_proj_w'].shape[1]
    proj = xc @ p['x_proj_w'].T
    dt = jax.nn.softplus(proj[..., :R] @ p['dt_proj_w'].T + p['dt_proj_b'])
    Bm, Cm = proj[..., R:R + N], proj[..., R + N:]
    A = -jnp.exp(p['A_log'])

    DN = Di * N
    af = A.T.reshape(1, DN)
    ebt = jnp.repeat(jnp.eye(N, dtype=jnp.float32), Di, axis=1)
    dv = p['D'].reshape(1, Di)
    dtf = dt.reshape(S, T * Di)
    dtxf = (dt * xc).reshape(S, T * Di)
    xcf = xc.reshape(S, T * Di)
    zf = z.reshape(S, T * Di)
    bmf = Bm.reshape(S, T * N)
    cmf = Cm.reshape(S, T * N)

    SB = 128
    body = functools.partial(_spe_scan_body, T, Di, N)
    seq = pl.BlockSpec((SB, T * Di), lambda i: (i, 0))
    g = pl.pallas_call(
        body,
        out_shape=jax.ShapeDtypeStruct((S, T * Di), jnp.float32),
        grid=(S // SB,),
        in_specs=[
            seq, seq, seq, seq,
            pl.BlockSpec((SB, T * N), lambda i: (i, 0)),
            pl.BlockSpec((SB, T * N), lambda i: (i, 0)),
            pl.BlockSpec((N, DN), lambda i: (0, 0)),
            pl.BlockSpec((1, DN), lambda i: (0, 0)),
            pl.BlockSpec((1, Di), lambda i: (0, 0)),
        ],
        out_specs=seq,
        compiler_params=pltpu.CompilerParams(
            dimension_semantics=("parallel",),
            vmem_limit_bytes=100 * 1024 * 1024,
        ),
        name="spe_scan",
    )(dtf, dtxf, xcf, zf, bmf, cmf, ebt, af, dv)
    return _mm(g.reshape(S * T, Di), p['out_proj_w'].T).reshape(S, T, D)


def _spa_branch(x, p):
    xp = _lsp(x, p['lsp'])
    xr = _prca(xp, p['prca'])
    B, H, W, C = xr.shape
    xf = _mamba_long(xr.reshape(B, H * W, C), p['mamba'])
    xr = xf.reshape(B, H, W, C)
    return jax.nn.silu(_gn(xr, p['gn_w'], p['gn_b'], 4)) + xp


def _spe_branch(x, p, token_num=4):
    xr = _prca(x, p['prca'])
    B, H, W, C = xr.shape
    gc = C // token_num
    xf = _mamba_short(xr.reshape(B * H * W, token_num, gc), p['mamba'])
    xr2 = xf.reshape(B, H, W, C)
    return jax.nn.silu(_gn(xr2, p['gn_w'], p['gn_b'], 4)) + x


def _gap(x):
    return x.mean((1, 2))


def _gate(x, w1, w2):
    return jax.nn.sigmoid(jax.nn.silu(_gap(x) @ w1.T) @ w2.T)[:, None, None, :]


def _bridge(spa, spe, p):
    spa_out = spa + p['gamma_spa'] * (_gate(spe, p['spa_w1'], p['spa_w2']) * spa)
    spe_out = spe + p['gamma_spe'] * (_gate(spa, p['spe_w1'], p['spe_w2']) * spe)
    return spa_out, spe_out


def _ccaf(spa, spe, p):
    logit = jnp.stack([_gap(spa) @ p['fc_spa'].T, _gap(spe) @ p['fc_spe'].T], axis=1)
    w = jax.nn.softmax(logit, axis=1)[:, :, None, None, :]
    g_cons = _gate(spa * spe, p['cons_w1'], p['cons_w2'])
    g_conf = _gate(jnp.abs(spa - spe), p['conf_w1'], p['conf_w2'])
    competitive = w[:, 0] * spa + w[:, 1] * spe
    consensus = g_cons * 0.5 * (spa + spe)
    return competitive + p['beta'] * consensus * (1.0 - g_conf)


def kernel(x, params):
    xh = x.transpose(0, 2, 3, 1)
    spa = _spa_branch(xh, params['spa'])
    spe = _spe_branch(xh, params['spe'])
    spa, spe = _bridge(spa, spe, params['bridge'])
    out = _ccaf(spa, spe, params['ccaf'])
    return out.transpose(0, 3, 1, 2)


# speedup vs baseline: 1.0991x; 1.0474x over previous
import functools
import math

import jax
import jax.numpy as jnp
from jax.experimental import pallas as pl
from jax.experimental.pallas import tpu as pltpu

_PAD = 128


def _mm_body(act, x_ref, w_ref, b_ref, o_ref):
    y = jnp.dot(x_ref[...], w_ref[...], preferred_element_type=jnp.float32) + b_ref[...]
    if act == 'silu':
        y = y * jax.nn.sigmoid(y)
    o_ref[...] = y


def _mm(x, w, b=None, act=None, bm=1152):
    M, K = x.shape
    N = w.shape[1]
    b2 = jnp.zeros((1, N), jnp.float32) if b is None else b.reshape(1, N)
    return pl.pallas_call(
        functools.partial(_mm_body, act),
        out_shape=jax.ShapeDtypeStruct((M, N), jnp.float32),
        grid=(M // bm,),
        in_specs=[pl.BlockSpec((bm, K), lambda i: (i, 0)),
                  pl.BlockSpec((K, N), lambda i: (0, 0)),
                  pl.BlockSpec((1, N), lambda i: (0, 0))],
        out_specs=pl.BlockSpec((bm, N), lambda i: (i, 0)),
        compiler_params=pltpu.CompilerParams(
            dimension_semantics=("parallel",),
            vmem_limit_bytes=100 * 1024 * 1024,
        ),
        name=f"mm_{N}",
    )(x, w, b2)


def _attn_chain_body(Wd, nlayers, x_ref, wqkv_ref, bqkv_ref, wdw_ref, bdw_ref,
                     temp_ref, wproj_ref, bproj_ref, o_ref, qpad_ref):
    n, C = x_ref.shape[1], x_ref.shape[2]
    C3 = 3 * C
    hd = C // 4

    qpad_ref[:_PAD] = jnp.zeros((_PAD, C3), jnp.float32)
    qpad_ref[_PAD + n:] = jnp.zeros((_PAD, C3), jnp.float32)

    rx = jax.lax.broadcasted_iota(jnp.int32, (n // Wd, Wd, C3), 1).reshape(n, C3)
    mp = jnp.where(rx < Wd - 2, 1.0, 0.0)
    mm_ = jnp.where(rx >= 2, 1.0, 0.0)

    y = x_ref[0]
    for l in range(nlayers):
        qkv = jax.lax.dot_general(y, wqkv_ref[l], (((1,), (1,)), ((), ())),
                                  preferred_element_type=jnp.float32) + bqkv_ref[l]
        qpad_ref[_PAD:_PAD + n] = qkv
        wd = wdw_ref[l]
        acc0 = jnp.zeros((n, C3), jnp.float32)
        accm = jnp.zeros((n, C3), jnp.float32)
        accp = jnp.zeros((n, C3), jnp.float32)
        for ky in range(3):
            off = (ky - 1) * 2 * Wd
            accm = accm + wd[3 * ky:3 * ky + 1] * qpad_ref[_PAD + off - 2:_PAD + off - 2 + n]
            acc0 = acc0 + wd[3 * ky + 1:3 * ky + 2] * qpad_ref[_PAD + off:_PAD + off + n]
            accp = accp + wd[3 * ky + 2:3 * ky + 3] * qpad_ref[_PAD + off + 2:_PAD + off + 2 + n]
        conv = acc0 + mm_ * accm + mp * accp + bdw_ref[l]

        outs = []
        for h in range(4):
            qh = conv[:, h * hd:(h + 1) * hd]
            kh = conv[:, C + h * hd:C + (h + 1) * hd]
            vh = conv[:, 2 * C + h * hd:2 * C + (h + 1) * hd]
            qn = qh * jax.lax.rsqrt(jnp.maximum(jnp.sum(qh * qh, axis=0, keepdims=True), 1e-24))
            kn = kh * jax.lax.rsqrt(jnp.maximum(jnp.sum(kh * kh, axis=0, keepdims=True), 1e-24))
            att = jax.lax.dot_general(qn, kn, (((0,), (0,)), ((), ())),
                                      preferred_element_type=jnp.float32)
            att = att * temp_ref[l, h]
            att = att - jnp.max(att, axis=-1, keepdims=True)
            att = jnp.exp(att)
            att = att / jnp.sum(att, axis=-1, keepdims=True)
            outs.append(jax.lax.dot_general(vh, att, (((1,), (1,)), ((), ())),
                                            preferred_element_type=jnp.float32))
        hall = jnp.concatenate(outs, axis=-1)
        y = jax.lax.dot_general(hall, wproj_ref[l], (((1,), (1,)), ((), ())),
                                preferred_element_type=jnp.float32) + bproj_ref[l]
    o_ref[0] = y


def _attn_chain(x, aps, Wd):
    B, n, C = x.shape
    C3 = 3 * C
    L = len(aps)
    wqkv = jnp.stack([ap['qkv_w'][:, :, 0, 0] for ap in aps])
    bqkv = jnp.stack([ap['qkv_b'][None, :] for ap in aps])
    wdw = jnp.stack([ap['dw_w'][:, 0].reshape(C3, 9).T for ap in aps])
    bdw = jnp.stack([ap['dw_b'][None, :] for ap in aps])
    temp = jnp.stack([ap['temp'][:, 0, 0] for ap in aps])
    wproj = jnp.stack([ap['proj_w'][:, :, 0, 0] for ap in aps])
    bproj = jnp.stack([ap['proj_b'][None, :] for ap in aps])

    body = functools.partial(_attn_chain_body, Wd, L)
    return pl.pallas_call(
        body,
        out_shape=jax.ShapeDtypeStruct((B, n, C), jnp.float32),
        grid=(B,),
        in_specs=[
            pl.BlockSpec((1, n, C), lambda b: (b, 0, 0)),
            pl.BlockSpec((L, C3, C), lambda b: (0, 0, 0)),
            pl.BlockSpec((L, 1, C3), lambda b: (0, 0, 0)),
            pl.BlockSpec((L, 9, C3), lambda b: (0, 0, 0)),
            pl.BlockSpec((L, 1, C3), lambda b: (0, 0, 0)),
            pl.BlockSpec((L, 4), lambda b: (0, 0), memory_space=pltpu.SMEM),
            pl.BlockSpec((L, C, C), lambda b: (0, 0, 0)),
            pl.BlockSpec((L, 1, C), lambda b: (0, 0, 0)),
        ],
        out_specs=pl.BlockSpec((1, n, C), lambda b: (b, 0, 0)),
        scratch_shapes=[pltpu.VMEM((n + 2 * _PAD, C3), jnp.float32)],
        compiler_params=pltpu.CompilerParams(
            dimension_semantics=("parallel",),
            vmem_limit_bytes=110 * 1024 * 1024,
        ),
        name=f"attn_chain_{Wd}",
    )(x, wqkv, bqkv, wdw, bdw, temp, wproj, bproj)


def _gn(x, w, b, groups, eps=1e-5):
    B, H, W, C = x.shape
    xg = x.reshape(B, H * W, groups, C // groups)
    m = xg.mean((1, 3), keepdims=True)
    v = ((xg - m) ** 2).mean((1, 3), keepdims=True)
    xg = (xg - m) * jax.lax.rsqrt(v + eps)
    return xg.reshape(B, H, W, C) * w + b


def _lsp(x, p):
    B, H, W, C = x.shape
    local = jax.lax.conv_general_dilated(
        x, p['dw_w'].transpose(2, 3, 1, 0), (1, 1), [(1, 1), (1, 1)],
        feature_group_count=C, dimension_numbers=('NHWC', 'HWIO', 'NHWC'))
    local = local + p['dw_b']
    x2d = x.reshape(B * H * W, C)
    g1 = _mm(x2d, p['g1_w'][:, :, 0, 0].T, p['g1_b'], act='silu')
    gate = jax.nn.sigmoid(g1 @ p['g2_w'][:, :, 0, 0].T + p['g2_b'])
    lg = local.reshape(B * H * W, C) * gate
    out = _mm(lg, p['pw_w'][:, :, 0, 0].T, p['pw_b']).reshape(B, H, W, C)
    return jax.nn.silu(_gn(out, p['gn_w'], p['gn_b'], 4)) + x


def _avgpool(x, k):
    B, H, W, C = x.shape
    return x.reshape(B, H // k, k, W // k, k, C).mean((2, 4))


def _prca(x, p):
    B, H, W, C = x.shape
    outs = []
    for i in range(len(p['attn'])):
        o = x if i == 0 else _avgpool(x, 2 ** i)
        Wd = W // (2 ** i)
        o3 = _attn_chain(o.reshape(B, Wd * Wd, C), p['attn'][i], Wd)
        o = o3.reshape(B, Wd, Wd, C)
        if i > 0:
            o = jax.image.resize(o, (B, H, W, C), 'bilinear')
        outs.append(o)
    cat = jnp.concatenate([o.reshape(B * H * W, C) for o in outs], axis=-1)
    out = _mm(cat, p['proj_w'][:, :, 0, 0].T, p['proj_b'])
    return out.reshape(B, H, W, C)


_Q = 64
_NST = 16


def _scan_body(dt_ref, dtx_ref, xc_ref, z_ref, bmd_ref, cmd_ref,
               rep_ref, arep_ref, dvec_ref, ow_ref, o_ref, hc_ref, H_ref):
    c = pl.program_id(1)

    @pl.when(c == 0)
    def _():
        hc_ref[...] = jnp.zeros_like(hc_ref)

    dtc = dt_ref[0]
    dt_bc = jnp.dot(rep_ref[...], dtc, preferred_element_type=jnp.float32)
    a = jnp.exp(dt_bc * arep_ref[...])
    bmat = jnp.dot(bmd_ref[0, 0], dtx_ref[0], preferred_element_type=jnp.float32)

    h = hc_ref[...]
    for s in range(_Q):
        h = a[s * _NST:(s + 1) * _NST] * h + bmat[s * _NST:(s + 1) * _NST]
        H_ref[s * _NST:(s + 1) * _NST, :] = h
    hc_ref[...] = h

    y = jnp.dot(cmd_ref[0, 0], H_ref[...], preferred_element_type=jnp.float32)
    y = y + xc_ref[0] * dvec_ref[...]
    z = z_ref[0]
    y = y * (z * jax.nn.sigmoid(z))
    o_ref[0] = jnp.dot(y, ow_ref[...], preferred_element_type=jnp.float32)


def _mamba_scan_pallas(dt, xc, z, Bm, Cm, A, Dvec, out_w):
    B, L, Di = dt.shape
    N = Bm.shape[-1]
    Q = _Q
    NC = L // Q
    QN = Q * N

    eyeQ = jnp.eye(Q, dtype=jnp.float32)
    Bm_r = Bm.reshape(B, NC, Q, N)
    Cm_r = Cm.reshape(B, NC, Q, N)
    bmdiag = (Bm_r[:, :, :, :, None] * eyeQ[None, None, :, None, :]).reshape(B, NC, QN, Q)
    cmdiag = (Cm_r[:, :, :, None, :] * eyeQ[None, None, :, :, None]).reshape(B, NC, Q, QN)
    rep = jnp.repeat(eyeQ, N, axis=0)
    arep = jnp.tile(A.T, (Q, 1))
    dtx = dt * xc
    dvec = Dvec[None, :]
    owT = out_w.T
    D = out_w.shape[0]

    grid = (B, NC)
    seq3 = pl.BlockSpec((1, Q, Di), lambda b, c: (b, c, 0))
    out = pl.pallas_call(
        _scan_body,
        out_shape=jax.ShapeDtypeStruct((B, L, D), jnp.float32),
        grid=grid,
        in_specs=[
            seq3,
            seq3,
            seq3,
            seq3,
            pl.BlockSpec((1, 1, QN, Q), lambda b, c: (b, c, 0, 0)),
            pl.BlockSpec((1, 1, Q, QN), lambda b, c: (b, c, 0, 0)),
            pl.BlockSpec((QN, Q), lambda b, c: (0, 0)),
            pl.BlockSpec((QN, Di), lambda b, c: (0, 0)),
            pl.BlockSpec((1, Di), lambda b, c: (0, 0)),
            pl.BlockSpec((Di, D), lambda b, c: (0, 0)),
        ],
        out_specs=pl.BlockSpec((1, Q, D), lambda b, c: (b, c, 0)),
        scratch_shapes=[
            pltpu.VMEM((N, Di), jnp.float32),
            pltpu.VMEM((QN, Di), jnp.float32),
        ],
        compiler_params=pltpu.CompilerParams(
            dimension_semantics=("parallel", "arbitrary"),
            vmem_limit_bytes=100 * 1024 * 1024,
        ),
        name="mamba_scan",
    )(dt, dtx, xc, z, bmdiag, cmdiag, rep, arep, dvec, owT)
    return out


def _mamba_long(x, p, N=16):
    B, L, D = x.shape
    xz = _mm(x.reshape(B * L, D), p['in_proj_w'].T).reshape(B, L, -1)
    Di = xz.shape[-1] // 2
    xin, z = xz[..., :Di], xz[..., Di:]
    kw = p['conv_w'].shape[-1]
    xc = jax.lax.conv_general_dilated(xin, p['conv_w'].transpose(2, 1, 0), (1,), [(kw - 1, 0)],
                                      feature_group_count=Di, dimension_numbers=('NWC', 'WIO', 'NWC'))
    xc = jax.nn.silu(xc + p['conv_b'])
    R = p['dt_proj_w'].shape[1]
    proj = xc @ p['x_proj_w'].T
    dt = jax.nn.softplus(proj[..., :R] @ p['dt_proj_w'].T + p['dt_proj_b'])
    Bm, Cm = proj[..., R:R + N], proj[..., R + N:]
    A = -jnp.exp(p['A_log'])
    return _mamba_scan_pallas(dt, xc, z, Bm, Cm, A, p['D'], p['out_proj_w'])


def _spe_scan_body(T, Di, N, dt_ref, dtx_ref, xc_ref, z_ref, bmc_ref, cmc_ref,
                   ebt_ref, af_ref, dv_ref, o_ref):
    SB = dt_ref.shape[0]
    DN = Di * N
    h = jnp.zeros((SB, DN), jnp.float32)
    for t in range(T):
        bex = jnp.dot(bmc_ref[:, t * N:(t + 1) * N], ebt_ref[...],
                      preferred_element_type=jnp.float32)
        cex = jnp.dot(cmc_ref[:, t * N:(t + 1) * N], ebt_ref[...],
                      preferred_element_type=jnp.float32)
        dt_t = dt_ref[:, t * Di:(t + 1) * Di]
        dtt = jnp.tile(jnp.concatenate([dt_t, dt_t], axis=1), (1, DN // (2 * Di)))
        a = jnp.exp(dtt * af_ref[...])
        dtx_t = dtx_ref[:, t * Di:(t + 1) * Di]
        dxt = jnp.tile(jnp.concatenate([dtx_t, dtx_t], axis=1), (1, DN // (2 * Di)))
        h = a * h + bex * dxt
        r = cex * h
        w = DN
        while w > Di:
            w //= 2
            r = r[:, :w] + r[:, w:2 * w]
        y = r + xc_ref[:, t * Di:(t + 1) * Di] * dv_ref[...]
        zt = z_ref[:, t * Di:(t + 1) * Di]
        o_ref[:, t * Di:(t + 1) * Di] = y * (zt * jax.nn.sigmoid(zt))


def _mamba_short(x, p, N=16):
    S, T, D = x.shape
    xz = _mm(x.reshape(S * T, D), p['in_proj_w'].T).reshape(S, T, -1)
    Di = xz.shape[-1] // 2
    xin, z = xz[..., :Di], xz[..., Di:]
    kw = p['conv_w'].shape[-1]
    xc = jax.lax.conv_general_dilated(xin, p['conv_w'].transpose(2, 1, 0), (1,), [(kw - 1, 0)],
                                      feature_group_count=Di, dimension_numbers=('NWC', 'WIO', 'NWC'))
    xc = jax.nn.silu(xc + p['conv_b'])
    R = p['dt_proj_w'].shape[1]
    proj = xc @ p['x_proj_w'].T
    dt = jax.nn.softplus(proj[..., :R] @ p['dt_proj_w'].T + p['dt_proj_b'])
    Bm, Cm = proj[..., R:R + N], proj[..., R + N:]
    A = -jnp.exp(p['A_log'])

    DN = Di * N
    af = A.T.reshape(1, DN)
    ebt = jnp.repeat(jnp.eye(N, dtype=jnp.float32), Di, axis=1)
    dv = p['D'].reshape(1, Di)
    dtf = dt.reshape(S, T * Di)
    dtxf = (dt * xc).reshape(S, T * Di)
    xcf = xc.reshape(S, T * Di)
    zf = z.reshape(S, T * Di)
    bmf = Bm.reshape(S, T * N)
    cmf = Cm.reshape(S, T * N)

    SB = 128
    body = functools.partial(_spe_scan_body, T, Di, N)
    seq = pl.BlockSpec((SB, T * Di), lambda i: (i, 0))
    g = pl.pallas_call(
        body,
        out_shape=jax.ShapeDtypeStruct((S, T * Di), jnp.float32),
        grid=(S // SB,),
        in_specs=[
            seq, seq, seq, seq,
            pl.BlockSpec((SB, T * N), lambda i: (i, 0)),
            pl.BlockSpec((SB, T * N), lambda i: (i, 0)),
            pl.BlockSpec((N, DN), lambda i: (0, 0)),
            pl.BlockSpec((1, DN), lambda i: (0, 0)),
            pl.BlockSpec((1, Di), lambda i: (0, 0)),
        ],
        out_specs=seq,
        compiler_params=pltpu.CompilerParams(
            dimension_semantics=("parallel",),
            vmem_limit_bytes=100 * 1024 * 1024,
        ),
        name="spe_scan",
    )(dtf, dtxf, xcf, zf, bmf, cmf, ebt, af, dv)
    return _mm(g.reshape(S * T, Di), p['out_proj_w'].T).reshape(S, T, D)


def _spa_branch(x, p):
    xp = _lsp(x, p['lsp'])
    xr = _prca(xp, p['prca'])
    B, H, W, C = xr.shape
    xf = _mamba_long(xr.reshape(B, H * W, C), p['mamba'])
    xr = xf.reshape(B, H, W, C)
    return jax.nn.silu(_gn(xr, p['gn_w'], p['gn_b'], 4)) + xp


def _spe_branch(x, p, token_num=4):
    xr = _prca(x, p['prca'])
    B, H, W, C = xr.shape
    gc = C // token_num
    xf = _mamba_short(xr.reshape(B * H * W, token_num, gc), p['mamba'])
    xr2 = xf.reshape(B, H, W, C)
    return jax.nn.silu(_gn(xr2, p['gn_w'], p['gn_b'], 4)) + x


def _gap(x):
    return x.mean((1, 2))


def _gate(x, w1, w2):
    return jax.nn.sigmoid(jax.nn.silu(_gap(x) @ w1.T) @ w2.T)[:, None, None, :]


def _bridge(spa, spe, p):
    spa_out = spa + p['gamma_spa'] * (_gate(spe, p['spa_w1'], p['spa_w2']) * spa)
    spe_out = spe + p['gamma_spe'] * (_gate(spa, p['spe_w1'], p['spe_w2']) * spe)
    return spa_out, spe_out


def _ccaf(spa, spe, p):
    logit = jnp.stack([_gap(spa) @ p['fc_spa'].T, _gap(spe) @ p['fc_spe'].T], axis=1)
    w = jax.nn.softmax(logit, axis=1)[:, :, None, None, :]
    g_cons = _gate(spa * spe, p['cons_w1'], p['cons_w2'])
    g_conf = _gate(jnp.abs(spa - spe), p['conf_w1'], p['conf_w2'])
    competitive = w[:, 0] * spa + w[:, 1] * spe
    consensus = g_cons * 0.5 * (spa + spe)
    return competitive + p['beta'] * consensus * (1.0 - g_conf)


def kernel(x, params):
    xh = x.transpose(0, 2, 3, 1)
    spa = _spa_branch(xh, params['spa'])
    spe = _spe_branch(xh, params['spe'])
    spa, spe = _bridge(spa, spe, params['bridge'])
    out = _ccaf(spa, spe, params['ccaf'])
    return out.transpose(0, 3, 1, 2)


# speedup vs baseline: 1.1541x; 1.0500x over previous
import functools
import math

import jax
import jax.numpy as jnp
from jax.experimental import pallas as pl
from jax.experimental.pallas import tpu as pltpu

_PAD = 128


def _mm_body(act, x_ref, w_ref, b_ref, o_ref):
    y = jnp.dot(x_ref[...], w_ref[...], preferred_element_type=jnp.float32) + b_ref[...]
    if act == 'silu':
        y = y * jax.nn.sigmoid(y)
    o_ref[...] = y


def _mm(x, w, b=None, act=None, bm=2304):
    M, K = x.shape
    N = w.shape[1]
    b2 = jnp.zeros((1, N), jnp.float32) if b is None else b.reshape(1, N)
    return pl.pallas_call(
        functools.partial(_mm_body, act),
        out_shape=jax.ShapeDtypeStruct((M, N), jnp.float32),
        grid=(M // bm,),
        in_specs=[pl.BlockSpec((bm, K), lambda i: (i, 0)),
                  pl.BlockSpec((K, N), lambda i: (0, 0)),
                  pl.BlockSpec((1, N), lambda i: (0, 0))],
        out_specs=pl.BlockSpec((bm, N), lambda i: (i, 0)),
        compiler_params=pltpu.CompilerParams(
            dimension_semantics=("parallel",),
            vmem_limit_bytes=100 * 1024 * 1024,
        ),
        name=f"mm_{N}",
    )(x, w, b2)


def _attn_chain_body(Wd, nlayers, x_ref, wqkv_ref, bqkv_ref, wdw_ref, bdw_ref,
                     temp_ref, wproj_ref, bproj_ref, o_ref, qpad_ref):
    n, C = x_ref.shape[1], x_ref.shape[2]
    C3 = 3 * C
    hd = C // 4

    qpad_ref[:_PAD] = jnp.zeros((_PAD, C3), jnp.float32)
    qpad_ref[_PAD + n:] = jnp.zeros((_PAD, C3), jnp.float32)

    rx = jax.lax.broadcasted_iota(jnp.int32, (n // Wd, Wd, C3), 1).reshape(n, C3)
    mp = jnp.where(rx < Wd - 2, 1.0, 0.0)
    mm_ = jnp.where(rx >= 2, 1.0, 0.0)

    y = x_ref[0]
    for l in range(nlayers):
        qkv = jax.lax.dot_general(y, wqkv_ref[l], (((1,), (1,)), ((), ())),
                                  preferred_element_type=jnp.float32) + bqkv_ref[l]
        qpad_ref[_PAD:_PAD + n] = qkv
        wd = wdw_ref[l]
        acc0 = jnp.zeros((n, C3), jnp.float32)
        accm = jnp.zeros((n, C3), jnp.float32)
        accp = jnp.zeros((n, C3), jnp.float32)
        for ky in range(3):
            off = (ky - 1) * 2 * Wd
            accm = accm + wd[3 * ky:3 * ky + 1] * qpad_ref[_PAD + off - 2:_PAD + off - 2 + n]
            acc0 = acc0 + wd[3 * ky + 1:3 * ky + 2] * qpad_ref[_PAD + off:_PAD + off + n]
            accp = accp + wd[3 * ky + 2:3 * ky + 3] * qpad_ref[_PAD + off + 2:_PAD + off + 2 + n]
        conv = acc0 + mm_ * accm + mp * accp + bdw_ref[l]

        outs = []
        for h in range(4):
            qh = conv[:, h * hd:(h + 1) * hd]
            kh = conv[:, C + h * hd:C + (h + 1) * hd]
            vh = conv[:, 2 * C + h * hd:2 * C + (h + 1) * hd]
            qn = qh * jax.lax.rsqrt(jnp.maximum(jnp.sum(qh * qh, axis=0, keepdims=True), 1e-24))
            kn = kh * jax.lax.rsqrt(jnp.maximum(jnp.sum(kh * kh, axis=0, keepdims=True), 1e-24))
            att = jax.lax.dot_general(qn, kn, (((0,), (0,)), ((), ())),
                                      preferred_element_type=jnp.float32)
            att = att * temp_ref[l, h]
            att = att - jnp.max(att, axis=-1, keepdims=True)
            att = jnp.exp(att)
            att = att / jnp.sum(att, axis=-1, keepdims=True)
            outs.append(jax.lax.dot_general(vh, att, (((1,), (1,)), ((), ())),
                                            preferred_element_type=jnp.float32))
        hall = jnp.concatenate(outs, axis=-1)
        y = jax.lax.dot_general(hall, wproj_ref[l], (((1,), (1,)), ((), ())),
                                preferred_element_type=jnp.float32) + bproj_ref[l]
    o_ref[0] = y


def _attn_chain(x, aps, Wd):
    B, n, C = x.shape
    C3 = 3 * C
    L = len(aps)
    wqkv = jnp.stack([ap['qkv_w'][:, :, 0, 0] for ap in aps])
    bqkv = jnp.stack([ap['qkv_b'][None, :] for ap in aps])
    wdw = jnp.stack([ap['dw_w'][:, 0].reshape(C3, 9).T for ap in aps])
    bdw = jnp.stack([ap['dw_b'][None, :] for ap in aps])
    temp = jnp.stack([ap['temp'][:, 0, 0] for ap in aps])
    wproj = jnp.stack([ap['proj_w'][:, :, 0, 0] for ap in aps])
    bproj = jnp.stack([ap['proj_b'][None, :] for ap in aps])

    body = functools.partial(_attn_chain_body, Wd, L)
    return pl.pallas_call(
        body,
        out_shape=jax.ShapeDtypeStruct((B, n, C), jnp.float32),
        grid=(B,),
        in_specs=[
            pl.BlockSpec((1, n, C), lambda b: (b, 0, 0)),
            pl.BlockSpec((L, C3, C), lambda b: (0, 0, 0)),
            pl.BlockSpec((L, 1, C3), lambda b: (0, 0, 0)),
            pl.BlockSpec((L, 9, C3), lambda b: (0, 0, 0)),
            pl.BlockSpec((L, 1, C3), lambda b: (0, 0, 0)),
            pl.BlockSpec((L, 4), lambda b: (0, 0), memory_space=pltpu.SMEM),
            pl.BlockSpec((L, C, C), lambda b: (0, 0, 0)),
            pl.BlockSpec((L, 1, C), lambda b: (0, 0, 0)),
        ],
        out_specs=pl.BlockSpec((1, n, C), lambda b: (b, 0, 0)),
        scratch_shapes=[pltpu.VMEM((n + 2 * _PAD, C3), jnp.float32)],
        compiler_params=pltpu.CompilerParams(
            dimension_semantics=("parallel",),
            vmem_limit_bytes=110 * 1024 * 1024,
        ),
        name=f"attn_chain_{Wd}",
    )(x, wqkv, bqkv, wdw, bdw, temp, wproj, bproj)


def _gn(x, w, b, groups, eps=1e-5):
    B, H, W, C = x.shape
    xg = x.reshape(B, H * W, groups, C // groups)
    m = xg.mean((1, 3), keepdims=True)
    v = ((xg - m) ** 2).mean((1, 3), keepdims=True)
    xg = (xg - m) * jax.lax.rsqrt(v + eps)
    return xg.reshape(B, H, W, C) * w + b


def _lsp(x, p):
    B, H, W, C = x.shape
    local = jax.lax.conv_general_dilated(
        x, p['dw_w'].transpose(2, 3, 1, 0), (1, 1), [(1, 1), (1, 1)],
        feature_group_count=C, dimension_numbers=('NHWC', 'HWIO', 'NHWC'))
    local = local + p['dw_b']
    x2d = x.reshape(B * H * W, C)
    g1 = _mm(x2d, p['g1_w'][:, :, 0, 0].T, p['g1_b'], act='silu')
    gate = jax.nn.sigmoid(g1 @ p['g2_w'][:, :, 0, 0].T + p['g2_b'])
    lg = local.reshape(B * H * W, C) * gate
    out = _mm(lg, p['pw_w'][:, :, 0, 0].T, p['pw_b']).reshape(B, H, W, C)
    return jax.nn.silu(_gn(out, p['gn_w'], p['gn_b'], 4)) + x


def _avgpool(x, k):
    B, H, W, C = x.shape
    return x.reshape(B, H // k, k, W // k, k, C).mean((2, 4))


def _prca(x, p):
    B, H, W, C = x.shape
    outs = []
    for i in range(len(p['attn'])):
        o = x if i == 0 else _avgpool(x, 2 ** i)
        Wd = W // (2 ** i)
        o3 = _attn_chain(o.reshape(B, Wd * Wd, C), p['attn'][i], Wd)
        o = o3.reshape(B, Wd, Wd, C)
        if i > 0:
            o = jax.image.resize(o, (B, H, W, C), 'bilinear')
        outs.append(o)
    cat = jnp.concatenate([o.reshape(B * H * W, C) for o in outs], axis=-1)
    out = _mm(cat, p['proj_w'][:, :, 0, 0].T, p['proj_b'])
    return out.reshape(B, H, W, C)


_Q = 128
_NST = 16


def _scan_body(dt_ref, dtx_ref, xc_ref, z_ref, bmd_ref, cmd_ref,
               rep_ref, arep_ref, dvec_ref, ow_ref, o_ref, hc_ref, H_ref):
    c = pl.program_id(1)

    @pl.when(c == 0)
    def _():
        hc_ref[...] = jnp.zeros_like(hc_ref)

    dtc = dt_ref[0]
    dt_bc = jnp.dot(rep_ref[...], dtc, preferred_element_type=jnp.float32)
    a = jnp.exp(dt_bc * arep_ref[...])
    bmat = jnp.dot(bmd_ref[0, 0], dtx_ref[0], preferred_element_type=jnp.float32)

    h = hc_ref[...]
    for s in range(_Q):
        h = a[s * _NST:(s + 1) * _NST] * h + bmat[s * _NST:(s + 1) * _NST]
        H_ref[s * _NST:(s + 1) * _NST, :] = h
    hc_ref[...] = h

    y = jnp.dot(cmd_ref[0, 0], H_ref[...], preferred_element_type=jnp.float32)
    y = y + xc_ref[0] * dvec_ref[...]
    z = z_ref[0]
    y = y * (z * jax.nn.sigmoid(z))
    o_ref[0] = jnp.dot(y, ow_ref[...], preferred_element_type=jnp.float32)


def _mamba_scan_pallas(dt, xc, z, Bm, Cm, A, Dvec, out_w):
    B, L, Di = dt.shape
    N = Bm.shape[-1]
    Q = _Q
    NC = L // Q
    QN = Q * N

    eyeQ = jnp.eye(Q, dtype=jnp.float32)
    Bm_r = Bm.reshape(B, NC, Q, N)
    Cm_r = Cm.reshape(B, NC, Q, N)
    bmdiag = (Bm_r[:, :, :, :, None] * eyeQ[None, None, :, None, :]).reshape(B, NC, QN, Q)
    cmdiag = (Cm_r[:, :, :, None, :] * eyeQ[None, None, :, :, None]).reshape(B, NC, Q, QN)
    rep = jnp.repeat(eyeQ, N, axis=0)
    arep = jnp.tile(A.T, (Q, 1))
    dtx = dt * xc
    dvec = Dvec[None, :]
    owT = out_w.T
    D = out_w.shape[0]

    grid = (B, NC)
    seq3 = pl.BlockSpec((1, Q, Di), lambda b, c: (b, c, 0))
    out = pl.pallas_call(
        _scan_body,
        out_shape=jax.ShapeDtypeStruct((B, L, D), jnp.float32),
        grid=grid,
        in_specs=[
            seq3,
            seq3,
            seq3,
            seq3,
            pl.BlockSpec((1, 1, QN, Q), lambda b, c: (b, c, 0, 0)),
            pl.BlockSpec((1, 1, Q, QN), lambda b, c: (b, c, 0, 0)),
            pl.BlockSpec((QN, Q), lambda b, c: (0, 0)),
            pl.BlockSpec((QN, Di), lambda b, c: (0, 0)),
            pl.BlockSpec((1, Di), lambda b, c: (0, 0)),
            pl.BlockSpec((Di, D), lambda b, c: (0, 0)),
        ],
        out_specs=pl.BlockSpec((1, Q, D), lambda b, c: (b, c, 0)),
        scratch_shapes=[
            pltpu.VMEM((N, Di), jnp.float32),
            pltpu.VMEM((QN, Di), jnp.float32),
        ],
        compiler_params=pltpu.CompilerParams(
            dimension_semantics=("parallel", "arbitrary"),
            vmem_limit_bytes=100 * 1024 * 1024,
        ),
        name="mamba_scan",
    )(dt, dtx, xc, z, bmdiag, cmdiag, rep, arep, dvec, owT)
    return out


def _mamba_long(x, p, N=16):
    B, L, D = x.shape
    xz = _mm(x.reshape(B * L, D), p['in_proj_w'].T).reshape(B, L, -1)
    Di = xz.shape[-1] // 2
    xin, z = xz[..., :Di], xz[..., Di:]
    kw = p['conv_w'].shape[-1]
    xc = jax.lax.conv_general_dilated(xin, p['conv_w'].transpose(2, 1, 0), (1,), [(kw - 1, 0)],
                                      feature_group_count=Di, dimension_numbers=('NWC', 'WIO', 'NWC'))
    xc = jax.nn.silu(xc + p['conv_b'])
    R = p['dt_proj_w'].shape[1]
    proj = xc @ p['x_proj_w'].T
    dt = jax.nn.softplus(proj[..., :R] @ p['dt_proj_w'].T + p['dt_proj_b'])
    Bm, Cm = proj[..., R:R + N], proj[..., R + N:]
    A = -jnp.exp(p['A_log'])
    return _mamba_scan_pallas(dt, xc, z, Bm, Cm, A, p['D'], p['out_proj_w'])


def _spe_scan_body(T, Di, N, dt_ref, dtx_ref, xc_ref, z_ref, bmc_ref, cmc_ref,
                   ebt_ref, af_ref, dv_ref, o_ref):
    SB = dt_ref.shape[0]
    DN = Di * N
    h = jnp.zeros((SB, DN), jnp.float32)
    for t in range(T):
        bex = jnp.dot(bmc_ref[:, t * N:(t + 1) * N], ebt_ref[...],
                      preferred_element_type=jnp.float32)
        cex = jnp.dot(cmc_ref[:, t * N:(t + 1) * N], ebt_ref[...],
                      preferred_element_type=jnp.float32)
        dt_t = dt_ref[:, t * Di:(t + 1) * Di]
        dtt = jnp.tile(jnp.concatenate([dt_t, dt_t], axis=1), (1, DN // (2 * Di)))
        a = jnp.exp(dtt * af_ref[...])
        dtx_t = dtx_ref[:, t * Di:(t + 1) * Di]
        dxt = jnp.tile(jnp.concatenate([dtx_t, dtx_t], axis=1), (1, DN // (2 * Di)))
        h = a * h + bex * dxt
        r = cex * h
        w = DN
        while w > Di:
            w //= 2
            r = r[:, :w] + r[:, w:2 * w]
        y = r + xc_ref[:, t * Di:(t + 1) * Di] * dv_ref[...]
        zt = z_ref[:, t * Di:(t + 1) * Di]
        o_ref[:, t * Di:(t + 1) * Di] = y * (zt * jax.nn.sigmoid(zt))


def _mamba_short(x, p, N=16):
    S, T, D = x.shape
    xz = _mm(x.reshape(S * T, D), p['in_proj_w'].T).reshape(S, T, -1)
    Di = xz.shape[-1] // 2
    xin, z = xz[..., :Di], xz[..., Di:]
    kw = p['conv_w'].shape[-1]
    xc = jax.lax.conv_general_dilated(xin, p['conv_w'].transpose(2, 1, 0), (1,), [(kw - 1, 0)],
                                      feature_group_count=Di, dimension_numbers=('NWC', 'WIO', 'NWC'))
    xc = jax.nn.silu(xc + p['conv_b'])
    R = p['dt_proj_w'].shape[1]
    proj = xc @ p['x_proj_w'].T
    dt = jax.nn.softplus(proj[..., :R] @ p['dt_proj_w'].T + p['dt_proj_b'])
    Bm, Cm = proj[..., R:R + N], proj[..., R + N:]
    A = -jnp.exp(p['A_log'])

    DN = Di * N
    af = A.T.reshape(1, DN)
    ebt = jnp.repeat(jnp.eye(N, dtype=jnp.float32), Di, axis=1)
    dv = p['D'].reshape(1, Di)
    dtf = dt.reshape(S, T * Di)
    dtxf = (dt * xc).reshape(S, T * Di)
    xcf = xc.reshape(S, T * Di)
    zf = z.reshape(S, T * Di)
    bmf = Bm.reshape(S, T * N)
    cmf = Cm.reshape(S, T * N)

    SB = 128
    body = functools.partial(_spe_scan_body, T, Di, N)
    seq = pl.BlockSpec((SB, T * Di), lambda i: (i, 0))
    g = pl.pallas_call(
        body,
        out_shape=jax.ShapeDtypeStruct((S, T * Di), jnp.float32),
        grid=(S // SB,),
        in_specs=[
            seq, seq, seq, seq,
            pl.BlockSpec((SB, T * N), lambda i: (i, 0)),
            pl.BlockSpec((SB, T * N), lambda i: (i, 0)),
            pl.BlockSpec((N, DN), lambda i: (0, 0)),
            pl.BlockSpec((1, DN), lambda i: (0, 0)),
            pl.BlockSpec((1, Di), lambda i: (0, 0)),
        ],
        out_specs=seq,
        compiler_params=pltpu.CompilerParams(
            dimension_semantics=("parallel",),
            vmem_limit_bytes=100 * 1024 * 1024,
        ),
        name="spe_scan",
    )(dtf, dtxf, xcf, zf, bmf, cmf, ebt, af, dv)
    return _mm(g.reshape(S * T, Di), p['out_proj_w'].T).reshape(S, T, D)


def _spa_branch(x, p):
    xp = _lsp(x, p['lsp'])
    xr = _prca(xp, p['prca'])
    B, H, W, C = xr.shape
    xf = _mamba_long(xr.reshape(B, H * W, C), p['mamba'])
    xr = xf.reshape(B, H, W, C)
    return jax.nn.silu(_gn(xr, p['gn_w'], p['gn_b'], 4)) + xp


def _spe_branch(x, p, token_num=4):
    xr = _prca(x, p['prca'])
    B, H, W, C = xr.shape
    gc = C // token_num
    xf = _mamba_short(xr.reshape(B * H * W, token_num, gc), p['mamba'])
    xr2 = xf.reshape(B, H, W, C)
    return jax.nn.silu(_gn(xr2, p['gn_w'], p['gn_b'], 4)) + x


def _gap(x):
    return x.mean((1, 2))


def _gate(x, w1, w2):
    return jax.nn.sigmoid(jax.nn.silu(_gap(x) @ w1.T) @ w2.T)[:, None, None, :]


def _bridge(spa, spe, p):
    spa_out = spa + p['gamma_spa'] * (_gate(spe, p['spa_w1'], p['spa_w2']) * spa)
    spe_out = spe + p['gamma_spe'] * (_gate(spa, p['spe_w1'], p['spe_w2']) * spe)
    return spa_out, spe_out


def _ccaf(spa, spe, p):
    logit = jnp.stack([_gap(spa) @ p['fc_spa'].T, _gap(spe) @ p['fc_spe'].T], axis=1)
    w = jax.nn.softmax(logit, axis=1)[:, :, None, None, :]
    g_cons = _gate(spa * spe, p['cons_w1'], p['cons_w2'])
    g_conf = _gate(jnp.abs(spa - spe), p['conf_w1'], p['conf_w2'])
    competitive = w[:, 0] * spa + w[:, 1] * spe
    consensus = g_cons * 0.5 * (spa + spe)
    return competitive + p['beta'] * consensus * (1.0 - g_conf)


def kernel(x, params):
    xh = x.transpose(0, 2, 3, 1)
    spa = _spa_branch(xh, params['spa'])
    spe = _spe_branch(xh, params['spe'])
    spa, spe = _bridge(spa, spe, params['bridge'])
    out = _ccaf(spa, spe, params['ccaf'])
    return out.transpose(0, 3, 1, 2)
